# baseline (speedup 1.0000x reference)
import functools
import jax
import jax.numpy as jnp
from jax import lax
from jax.experimental import pallas as pl
from jax.experimental.pallas import tpu as pltpu
from jax.experimental.pallas import tpu_sc as plsc

N = 10000
NP = 10240
E = 320000
EP = 327680
F = 128
H = 16
NC = 2
NS = 16
NW = NC * NS
EB = 128
ROWS = EP // EB
RW = ROWS // NW
NPS = NP // NS


CH = 20
NCH = RW // CH
NBD = 20


def _sc_body(gather, src_hbm, dst_hbm, table_hbm, fill_hbm, out_hbm,
             src_idx, dst_idx, bufa, bufb, nbuf, acc,
             gsema, gsemb, ssema, ssemb):
    c = lax.axis_index("c")
    s = lax.axis_index("s")
    w = c * NS + s

    pltpu.sync_copy(fill_hbm.at[pl.ds(0, NPS)], nbuf)
    pltpu.sync_copy(nbuf, acc.at[pl.ds(s * NPS, NPS)])
    plsc.subcore_barrier()

    pltpu.sync_copy(dst_hbm.at[pl.ds(w * RW, RW)], dst_idx)
    if gather:
        pltpu.sync_copy(src_hbm.at[pl.ds(w * RW, RW)], src_idx)

    def fire_gathers(ci, buf, sem):
        def f(j, _):
            pltpu.async_copy(table_hbm.at[src_idx.at[ci * CH + j]],
                             buf.at[pl.ds(j * EB, EB)], sem)
            return _
        lax.fori_loop(0, CH, f, None)

    def fire_scatters(ci, buf, sem):
        def f(j, _):
            pltpu.async_copy(buf.at[pl.ds(j * EB, EB)],
                             acc.at[dst_idx.at[ci * CH + j]], sem, add=True)
            return _
        lax.fori_loop(0, CH, f, None)

    def wait_chunk(buf, sem):
        pltpu.make_async_copy(table_hbm.at[pl.ds(0, CH * EB)], buf, sem).wait()

    if gather:
        bufs = (bufa, bufb)
        gsems = (gsema, gsemb)
        ssems = (ssema, ssemb)
        fire_gathers(0, bufa, gsema)
        fire_gathers(1, bufb, gsemb)
        wait_chunk(bufa, gsema)
        fire_scatters(0, bufa, ssema)
        wait_chunk(bufb, gsemb)
        fire_scatters(1, bufb, ssemb)
        for ci in range(2, NCH):
            p = ci % 2
            wait_chunk(bufs[p], ssems[p])
            fire_gathers(ci, bufs[p], gsems[p])
            wait_chunk(bufs[p], gsems[p])
            fire_scatters(ci, bufs[p], ssems[p])
        wait_chunk(bufs[NCH % 2], ssems[NCH % 2])
        wait_chunk(bufs[(NCH + 1) % 2], ssems[(NCH + 1) % 2])
    else:
        pltpu.sync_copy(fill_hbm.at[pl.ds(NPS, EB)], bufa.at[pl.ds(0, EB)])
        ones = bufa.at[pl.ds(0, EB)]

        def fire_deg(ci):
            def f(j, _):
                pltpu.async_copy(ones, acc.at[dst_idx.at[ci * NBD + j]],
                                 ssema, add=True)
                return _
            lax.fori_loop(0, NBD, f, None)

        nchd = RW // NBD
        fire_deg(0)
        for ci in range(1, nchd):
            fire_deg(ci)
            wait_chunk(bufa, ssema)
        wait_chunk(bufa, ssema)

    plsc.subcore_barrier()

    pltpu.sync_copy(acc.at[pl.ds(s * NPS, NPS)], nbuf)
    pltpu.sync_copy(nbuf, out_hbm.at[pl.ds((c * NP) + s * NPS, NPS)])


def _make_sc_pass(gather):
    mesh = plsc.VectorSubcoreMesh(core_axis_name="c", subcore_axis_name="s")
    scratch = [
        pltpu.VMEM((RW, EB), jnp.int32),
        pltpu.VMEM((RW, EB), jnp.int32),
        pltpu.VMEM((CH * EB, H), jnp.float32),
        pltpu.VMEM((CH * EB, H), jnp.float32),
        pltpu.VMEM((NPS, H), jnp.float32),
        pltpu.VMEM_SHARED((NP, H), jnp.float32),
        pltpu.SemaphoreType.DMA,
        pltpu.SemaphoreType.DMA,
        pltpu.SemaphoreType.DMA,
        pltpu.SemaphoreType.DMA,
    ]
    return pl.kernel(
        functools.partial(_sc_body, gather),
        out_type=jax.ShapeDtypeStruct((NC * NP, H), jnp.float32),
        mesh=mesh,
        scratch_types=scratch,
        compiler_params=pltpu.CompilerParams(use_tc_tiling_on_sc=False),
        name="gcn_scatter" if gather else "gcn_degree",
    )


RB = 1024
GRID = NP // RB


def _tc1_body(x, w1, h0, h1, u_out, dinv_out):
    deg = h0[:, 0:1] + h1[:, 0:1] + 1.0
    dinv = lax.rsqrt(deg)
    h = jnp.dot(x[...], w1[...], preferred_element_type=jnp.float32)
    u_out[...] = h * dinv
    dinv_out[...] = dinv


def _tc2_body(q0, q1, u, dinv, b1, w2, g_out):
    s = q0[...] + q1[...] + u[...]
    l1 = jnp.maximum(dinv[...] * s + b1[...], 0.0)
    g = jnp.dot(l1, w2[...], preferred_element_type=jnp.float32)
    g_out[...] = g * dinv[...]


def _tc3_body(r0, r1, g, dinv, b2, o_out):
    o_out[...] = dinv[...] * (r0[...] + r1[...] + g[...]) + b2[...]


def _part_specs():
    return [
        pl.BlockSpec((RB, H), _row0),
        pl.BlockSpec((RB, H), _row1),
    ]


_row0 = lambda i: (i, 0)
_row1 = lambda i: (i + GRID, 0)
_full = lambda i: (0, 0)


def _tc1(x, w1, hist):
    return pl.pallas_call(
        _tc1_body,
        grid=(GRID,),
        in_specs=[
            pl.BlockSpec((RB, F), _row0),
            pl.BlockSpec((F, H), _full),
            *_part_specs(),
        ],
        out_specs=[
            pl.BlockSpec((RB, H), _row0),
            pl.BlockSpec((RB, 1), _row0),
        ],
        out_shape=[
            jax.ShapeDtypeStruct((NP, H), jnp.float32),
            jax.ShapeDtypeStruct((NP, 1), jnp.float32),
        ],
    )(x, w1, hist, hist)


def _tc2(q, u, dinv, b1, w2):
    return pl.pallas_call(
        _tc2_body,
        grid=(GRID,),
        in_specs=[
            *_part_specs(),
            pl.BlockSpec((RB, H), _row0),
            pl.BlockSpec((RB, 1), _row0),
            pl.BlockSpec((1, H), _full),
            pl.BlockSpec((H, H), _full),
        ],
        out_specs=pl.BlockSpec((RB, H), _row0),
        out_shape=jax.ShapeDtypeStruct((NP, H), jnp.float32),
    )(q, q, u, dinv, b1, w2)


def _tc3(r, g, dinv, b2):
    return pl.pallas_call(
        _tc3_body,
        grid=(GRID,),
        in_specs=[
            *_part_specs(),
            pl.BlockSpec((RB, H), _row0),
            pl.BlockSpec((RB, 1), _row0),
            pl.BlockSpec((1, H), _full),
        ],
        out_specs=pl.BlockSpec((RB, H), _row0),
        out_shape=jax.ShapeDtypeStruct((NP, H), jnp.float32),
    )(r, r, g, dinv, b2)


_hist_pass = _make_sc_pass(gather=False)
_msg_pass = _make_sc_pass(gather=True)


@jax.jit
def kernel(x, edge_index, W1, b1, W2, b2):
    pad = jnp.full((EP - E,), N, jnp.int32)
    src = jnp.concatenate([edge_index[0], pad]).reshape(ROWS, EB)
    dst = jnp.concatenate([edge_index[1], pad]).reshape(ROWS, EB)
    fill = jnp.concatenate(
        [jnp.zeros((NPS, H), jnp.float32), jnp.ones((EB, H), jnp.float32)])
    dummy_table = jnp.zeros((NP, H), jnp.float32)

    w2p = jnp.zeros((H, H), jnp.float32).at[:, :W2.shape[1]].set(W2)
    b1r = b1.reshape(1, H)
    b2p = jnp.zeros((1, H), jnp.float32).at[0, :b2.shape[0]].set(b2)

    hist = _hist_pass(src, dst, dummy_table, fill)
    u, dinv = _tc1(x, W1, hist)
    q = _msg_pass(src, dst, u, fill)
    g = _tc2(q, u, dinv, b1r, w2p)
    r = _msg_pass(src, dst, g, fill)
    out = _tc3(r, g, dinv, b2p)
    return out[:N, :b2.shape[0]]


# speedup vs baseline: 2.0121x; 2.0121x over previous
import jax
import jax.numpy as jnp
from jax import lax
from jax.experimental import pallas as pl
from jax.experimental.pallas import tpu as pltpu
from jax.experimental.pallas import tpu_sc as plsc

N = 10000
NP = 10240
E = 320000
EP = 327680
F = 128
H = 16
NC = 2
NS = 16
NW = NC * NS
EB = 128
ROWS = EP // EB
RW = ROWS // NW
NPS = NP // NS


NOP = RW // 8
NBUF = 4


def _sc_body(src_hbm, dst_hbm, table_hbm, fill_hbm, fones_hbm,
             out_hbm, src_idx, dst_idx, buf0, buf1, buf2, buf3, nbuf, acc,
             tbl, gsem0, gsem1, gsem2, gsem3, ssem0, ssem1, ssem2, ssem3):
    c = lax.axis_index("c")
    s = lax.axis_index("s")
    w = c * NS + s
    bufs = (buf0, buf1, buf2, buf3)
    gsems = (gsem0, gsem1, gsem2, gsem3)
    ssems = (ssem0, ssem1, ssem2, ssem3)

    pltpu.sync_copy(fill_hbm.at[pl.ds(0, NPS)], nbuf)
    pltpu.sync_copy(nbuf, acc.at[pl.ds(s * NPS, NPS)])
    pltpu.sync_copy(table_hbm.at[pl.ds(s * NPS, NPS)],
                    tbl.at[pl.ds(s * NPS, NPS)])
    plsc.subcore_barrier()

    pltpu.sync_copy(dst_hbm.at[pl.ds(w * NOP, NOP)], dst_idx)
    pltpu.sync_copy(src_hbm.at[pl.ds(w * NOP, NOP)], src_idx)

    def wait_one(buf, sem):
        pltpu.make_async_copy(fones_hbm, buf, sem).wait()

    def fire_g(j, p):
        pltpu.async_copy(tbl.at[src_idx.at[j]], bufs[p], gsems[p])

    def fire_s(j, p, sem):
        pltpu.async_copy(bufs[p], acc.at[dst_idx.at[j]], sem, add=True)

    fire_g(0, 0)
    fire_g(1, 1)
    for j in range(NOP):
        p = j % NBUF
        wait_one(bufs[p], gsems[p])
        fire_s(j, p, ssems[p])
        if j + 2 < NOP and j >= 2:
            pn = (j + 2) % NBUF
            wait_one(bufs[pn], ssems[pn])
            fire_g(j + 2, pn)
        elif j + 2 < NOP:
            fire_g(j + 2, (j + 2) % NBUF)
    for j in range(NOP - 4, NOP):
        wait_one(bufs[j % NBUF], ssems[j % NBUF])

    plsc.subcore_barrier()

    pltpu.sync_copy(acc.at[pl.ds(s * NPS, NPS)], nbuf)
    pltpu.sync_copy(nbuf, out_hbm.at[pl.ds((c * NP) + s * NPS, NPS)])


def _hist_body(dst_hbm, fill_hbm, fones_hbm, out_hbm,
               dst_idx, buf0, nbuf, acc, ssem0):
    c = lax.axis_index("c")
    s = lax.axis_index("s")
    w = c * NS + s

    pltpu.sync_copy(fill_hbm.at[pl.ds(0, NPS)], nbuf)
    pltpu.sync_copy(nbuf, acc.at[pl.ds(s * NPS, NPS)])
    plsc.subcore_barrier()

    pltpu.sync_copy(dst_hbm.at[pl.ds(w * NOP, NOP)], dst_idx)
    pltpu.sync_copy(fones_hbm, buf0)
    for j in range(NOP):
        pltpu.async_copy(buf0, acc.at[dst_idx.at[j]], ssem0, add=True)
    for j in range(NOP):
        pltpu.make_async_copy(fones_hbm, buf0, ssem0).wait()

    plsc.subcore_barrier()
    pltpu.sync_copy(acc.at[pl.ds(s * NPS, NPS)], nbuf)
    pltpu.sync_copy(nbuf, out_hbm.at[pl.ds((c * NP) + s * NPS, NPS)])

def _make_msg_pass():
    mesh = plsc.VectorSubcoreMesh(core_axis_name="c", subcore_axis_name="s")
    scratch = [
        pltpu.VMEM((NOP, 8 * EB), jnp.int32),
        pltpu.VMEM((NOP, 8 * EB), jnp.int32),
        pltpu.VMEM((8 * EB, H), jnp.float32),
        pltpu.VMEM((8 * EB, H), jnp.float32),
        pltpu.VMEM((8 * EB, H), jnp.float32),
        pltpu.VMEM((8 * EB, H), jnp.float32),
        pltpu.VMEM((NPS, H), jnp.float32),
        pltpu.VMEM_SHARED((NP, H), jnp.float32),
        pltpu.VMEM_SHARED((NP, H), jnp.float32),
        pltpu.SemaphoreType.DMA,
        pltpu.SemaphoreType.DMA,
        pltpu.SemaphoreType.DMA,
        pltpu.SemaphoreType.DMA,
        pltpu.SemaphoreType.DMA,
        pltpu.SemaphoreType.DMA,
        pltpu.SemaphoreType.DMA,
        pltpu.SemaphoreType.DMA,
    ]
    return pl.kernel(
        _sc_body,
        out_type=jax.ShapeDtypeStruct((NC * NP, H), jnp.float32),
        mesh=mesh,
        scratch_types=scratch,
        compiler_params=pltpu.CompilerParams(use_tc_tiling_on_sc=False),
        name="gcn_scatter",
    )


def _make_hist_pass():
    mesh = plsc.VectorSubcoreMesh(core_axis_name="c", subcore_axis_name="s")
    scratch = [
        pltpu.VMEM((NOP, 8 * EB), jnp.int32),
        pltpu.VMEM((8 * EB, H), jnp.float32),
        pltpu.VMEM((NPS, H), jnp.float32),
        pltpu.VMEM_SHARED((NP, H), jnp.float32),
        pltpu.SemaphoreType.DMA,
    ]
    return pl.kernel(
        _hist_body,
        out_type=jax.ShapeDtypeStruct((NC * NP, H), jnp.float32),
        mesh=mesh,
        scratch_types=scratch,
        compiler_params=pltpu.CompilerParams(use_tc_tiling_on_sc=False),
        name="gcn_degree",
    )


PR = NP // 8


def _tc1a_body(x, w1, h_out):
    h_out[...] = jnp.dot(x[...], w1[...], preferred_element_type=jnp.float32)


def _tc1b_body(h, hist, u_out, dinv_out):
    dinv = lax.rsqrt(hist[0:PR, :] + hist[PR:2 * PR, :] + 1.0)
    u_out[...] = h[...] * dinv
    dinv_out[...] = dinv


def _tc2_body(q, u, dinv, b1, w2, g_out):
    s = q[0:PR, :] + q[PR:2 * PR, :] + u[...]
    l1 = jnp.maximum(dinv[...] * s + b1[...], 0.0)
    g = jnp.dot(l1, w2[...], preferred_element_type=jnp.float32)
    g_out[...] = g * dinv[...]


def _tc3_body(r, g, dinv, b2, o_out):
    o_out[...] = dinv[...] * (r[0:PR, :] + r[PR:2 * PR, :] + g[...]) + b2[...]


def _tc1a(x2, w1bd):
    return pl.pallas_call(
        _tc1a_body,
        out_shape=jax.ShapeDtypeStruct((PR, 128), jnp.float32),
    )(x2, w1bd)


def _tc1b(h, hist2):
    return pl.pallas_call(
        _tc1b_body,
        out_shape=[
            jax.ShapeDtypeStruct((PR, 128), jnp.float32),
            jax.ShapeDtypeStruct((PR, 128), jnp.float32),
        ],
    )(h, hist2)


def _tc2(q2, u, dinv, b1p, w2bd):
    return pl.pallas_call(
        _tc2_body,
        out_shape=jax.ShapeDtypeStruct((PR, 128), jnp.float32),
    )(q2, u, dinv, b1p, w2bd)


def _tc3(r2, g, dinv, b2p):
    return pl.pallas_call(
        _tc3_body,
        out_shape=jax.ShapeDtypeStruct((PR, 128), jnp.float32),
    )(r2, g, dinv, b2p)


_hist_pass = _make_hist_pass()
_msg_pass = _make_msg_pass()


@jax.jit
def kernel(x, edge_index, W1, b1, W2, b2):
    pad = jnp.full((EP - E,), N, jnp.int32)
    src = jnp.concatenate([edge_index[0], pad]).reshape(ROWS // 8, 8 * EB)
    dst = jnp.concatenate([edge_index[1], pad]).reshape(ROWS // 8, 8 * EB)
    fill = jnp.concatenate(
        [jnp.zeros((NPS, H), jnp.float32), jnp.ones((EB, H), jnp.float32)])
    fones = jnp.ones((8 * EB, H), jnp.float32)

    nc = b2.shape[0]
    x2 = jnp.pad(x, ((0, NP - N), (0, 0))).reshape(PR, 8 * F)
    w1bd = jnp.kron(jnp.eye(8, dtype=jnp.float32), W1)
    w2p = jnp.zeros((H, H), jnp.float32).at[:, :nc].set(W2)
    w2bd = jnp.kron(jnp.eye(8, dtype=jnp.float32), w2p)
    b1p = jnp.tile(b1, 8).reshape(1, 128)
    b2p = jnp.tile(jnp.pad(b2, (0, H - nc)), 8).reshape(1, 128)

    hist = _hist_pass(dst, fill, fones)
    h = _tc1a(x2, w1bd)
    u, dinv = _tc1b(h, hist.reshape(2 * PR, 128))
    q = _msg_pass(src, dst, u.reshape(NP, H), fill, fones)
    g = _tc2(q.reshape(2 * PR, 128), u, dinv, b1p, w2bd)
    r = _msg_pass(src, dst, g.reshape(NP, H), fill, fones)
    out = _tc3(r.reshape(2 * PR, 128), g, dinv, b2p)
    return out.reshape(NP, H)[:N, :nc]
